# baseline (speedup 1.0000x reference)
import functools

import jax
import jax.numpy as jnp
from jax import lax
from jax.experimental import pallas as pl
from jax.experimental.pallas import tpu as pltpu
from jax.experimental.pallas import tpu_sc as plsc

N = 10000
C_TC = 9472
C_SC = N - C_TC

L = 16
NW = 32
W = 32
NV = W // L
STEP = (C_SC - W) // (NW - 1)
RB = 200
NBLK = N // RB

BR = 400
TC_NBLK = N // BR
BIG = 2 ** 30


def _make_sc_kernel():
    mesh = plsc.VectorSubcoreMesh(core_axis_name="c", subcore_axis_name="s")

    @functools.partial(
        pl.kernel,
        out_type=(
            jax.ShapeDtypeStruct((1, C_SC), jnp.float32),
            jax.ShapeDtypeStruct((C_SC,), jnp.int32),
        ),
        mesh=mesh,
        compiler_params=pltpu.CompilerParams(use_tc_tiling_on_sc=False,
                                             needs_layout_passes=False),
        scratch_types=[
            pltpu.VMEM((RB, W), jnp.float32),
            pltpu.VMEM((RB, W), jnp.float32),
            pltpu.VMEM((RB, L), jnp.float32),
            pltpu.VMEM((RB, L), jnp.float32),
            pltpu.VMEM((W,), jnp.float32),
            pltpu.VMEM((W,), jnp.int32),
            pltpu.SemaphoreType.DMA,
            pltpu.SemaphoreType.DMA,
            pltpu.SemaphoreType.DMA,
            pltpu.SemaphoreType.DMA,
        ],
    )
    def sc_kernel(e_hbm, r_hbm, out_emb, out_idx,
                  buf0, buf1, ebuf0, ebuf1, val_v, idx_v,
                  sem0, sem1, esem0, esem1):
        cid = lax.axis_index("c")
        sid = lax.axis_index("s")
        w = sid * 2 + cid
        rel0 = pl.multiple_of(w * STEP, 8)
        c0 = pl.multiple_of(C_TC + w * STEP, 8)

        for j in range(NV):
            val_v[pl.ds(j * L, L)] = jnp.zeros((L,), jnp.float32)
            idx_v[pl.ds(j * L, L)] = jnp.zeros((L,), jnp.int32)

        bufs = (buf0, buf1)
        ebufs = (ebuf0, ebuf1)
        sems = (sem0, sem1)
        esems = (esem0, esem1)

        def start(b, k):
            pltpu.async_copy(
                r_hbm.at[pl.ds(b * RB, RB), pl.ds(c0, W)], bufs[k], sems[k])
            pltpu.async_copy(
                e_hbm.at[pl.ds(b * RB, RB), :], ebufs[k], esems[k])

        def wait(b, k):
            pltpu.make_async_copy(
                r_hbm.at[pl.ds(b * RB, RB), pl.ds(c0, W)], bufs[k],
                sems[k]).wait()
            pltpu.make_async_copy(
                e_hbm.at[pl.ds(b * RB, RB), :], ebufs[k],
                esems[k]).wait()

        start(0, 0)
        start(1, 1)

        def process(b, buf, ebuf):
            base = b * RB
            carry = tuple(
                val_v[pl.ds(j * L, L)] for j in range(NV)
            ) + tuple(
                idx_v[pl.ds(j * L, L)] for j in range(NV)
            )

            def row_body(r, cr, base=base, buf=buf, ebuf=ebuf):
                vals = list(cr[:NV])
                idxs = list(cr[NV:])
                ev = ebuf[r, :]
                ivec = jnp.full((L,), base, jnp.int32) + r
                for j in range(NV):
                    prod = buf[r, pl.ds(j * L, L)] * ev
                    m = prod > vals[j]
                    vals[j] = jnp.where(m, prod, vals[j])
                    idxs[j] = jnp.where(m, ivec, idxs[j])
                return tuple(vals) + tuple(idxs)

            carry = lax.fori_loop(0, RB, row_body, carry)
            for j in range(NV):
                val_v[pl.ds(j * L, L)] = carry[j]
                idx_v[pl.ds(j * L, L)] = carry[NV + j]

        def outer(g, acc):
            for k in range(2):
                b = 2 * g + k
                wait(b, k)
                process(b, bufs[k], ebufs[k])

                @pl.when(b + 2 < NBLK)
                def _(b=b, k=k):
                    start(b + 2, k)
            return acc

        lax.fori_loop(0, NBLK // 2, outer, 0)
        if NBLK % 2:
            wait(NBLK - 1, 0)
            process(NBLK - 1, bufs[0], ebufs[0])

        pltpu.sync_copy(val_v, out_emb.at[0, pl.ds(rel0, W)])
        pltpu.sync_copy(idx_v, out_idx.at[pl.ds(rel0, W)])

    return sc_kernel


def _tc_body(e_ref, r_ref, val_ref, idx_ref):
    i = pl.program_id(0)
    prod = r_ref[...] * e_ref[...]
    bmax = jnp.max(prod, axis=0, keepdims=True)
    rows = lax.broadcasted_iota(jnp.int32, (BR, C_TC), 0) + i * BR
    cand = jnp.where(prod == bmax, rows, BIG)
    barg = jnp.min(cand, axis=0, keepdims=True)

    @pl.when(i == 0)
    def _():
        val_ref[...] = jnp.zeros_like(val_ref)
        idx_ref[...] = jnp.zeros_like(idx_ref)

    m = bmax > val_ref[...]
    idx_ref[...] = jnp.where(m, barg, idx_ref[...])
    val_ref[...] = jnp.where(m, bmax, val_ref[...])


_tc_kernel = pl.pallas_call(
    _tc_body,
    grid=(TC_NBLK,),
    in_specs=[
        pl.BlockSpec((BR, 1), lambda i: (i, 0)),
        pl.BlockSpec((BR, C_TC), lambda i: (i, 0)),
    ],
    out_specs=[
        pl.BlockSpec((1, C_TC), lambda i: (0, 0)),
        pl.BlockSpec((1, C_TC), lambda i: (0, 0)),
    ],
    out_shape=[
        jax.ShapeDtypeStruct((1, C_TC), jnp.float32),
        jax.ShapeDtypeStruct((1, C_TC), jnp.int32),
    ],
)

_sc_kernel = _make_sc_kernel()


@jax.jit
def kernel(embedding, r_embedding):
    e_col = embedding.reshape(N, 1)
    e_exp = jnp.broadcast_to(e_col, (N, L))
    sc_val, sc_idx = _sc_kernel(e_exp, r_embedding)
    tc_val, tc_idx = _tc_kernel(e_col, r_embedding)
    new_embedding = jnp.concatenate([tc_val, sc_val], axis=1)
    r_argmax = jnp.concatenate([tc_idx.reshape(C_TC), sc_idx])
    return new_embedding, r_argmax


# speedup vs baseline: 1.0107x; 1.0107x over previous
import functools

import jax
import jax.numpy as jnp
from jax import lax
from jax.experimental import pallas as pl
from jax.experimental.pallas import tpu as pltpu
from jax.experimental.pallas import tpu_sc as plsc

N = 10000
C_TC = 9472
C_SC = N - C_TC

L = 16
NW = 32
W = 32
NV = W // L
STEP = (C_SC - W) // (NW - 1)
RB = 200
NBLK = N // RB
U = 8

BR = 400
TC_NBLK = N // BR
BIG = 2 ** 30


def _make_sc_kernel():
    mesh = plsc.VectorSubcoreMesh(core_axis_name="c", subcore_axis_name="s")

    @functools.partial(
        pl.kernel,
        out_type=(
            jax.ShapeDtypeStruct((1, C_SC), jnp.float32),
            jax.ShapeDtypeStruct((C_SC,), jnp.int32),
        ),
        mesh=mesh,
        compiler_params=pltpu.CompilerParams(use_tc_tiling_on_sc=False,
                                             needs_layout_passes=False),
        scratch_types=[
            pltpu.VMEM((RB, W), jnp.float32),
            pltpu.VMEM((RB, W), jnp.float32),
            pltpu.VMEM((RB, L), jnp.float32),
            pltpu.VMEM((RB, L), jnp.float32),
            pltpu.VMEM((W,), jnp.float32),
            pltpu.VMEM((W,), jnp.int32),
            pltpu.SemaphoreType.DMA,
            pltpu.SemaphoreType.DMA,
            pltpu.SemaphoreType.DMA,
            pltpu.SemaphoreType.DMA,
        ],
    )
    def sc_kernel(e_hbm, r_hbm, out_emb, out_idx,
                  buf0, buf1, ebuf0, ebuf1, val_v, idx_v,
                  sem0, sem1, esem0, esem1):
        cid = lax.axis_index("c")
        sid = lax.axis_index("s")
        w = sid * 2 + cid
        rel0 = pl.multiple_of(w * STEP, 8)
        c0 = pl.multiple_of(C_TC + w * STEP, 8)

        for j in range(NV):
            val_v[pl.ds(j * L, L)] = jnp.zeros((L,), jnp.float32)
            idx_v[pl.ds(j * L, L)] = jnp.zeros((L,), jnp.int32)

        bufs = (buf0, buf1)
        ebufs = (ebuf0, ebuf1)
        sems = (sem0, sem1)
        esems = (esem0, esem1)

        def start(b, k):
            pltpu.async_copy(
                r_hbm.at[pl.ds(b * RB, RB), pl.ds(c0, W)], bufs[k], sems[k])
            pltpu.async_copy(
                e_hbm.at[pl.ds(b * RB, RB), :], ebufs[k], esems[k])

        def wait(b, k):
            pltpu.make_async_copy(
                r_hbm.at[pl.ds(b * RB, RB), pl.ds(c0, W)], bufs[k],
                sems[k]).wait()
            pltpu.make_async_copy(
                e_hbm.at[pl.ds(b * RB, RB), :], ebufs[k],
                esems[k]).wait()

        start(0, 0)
        start(1, 1)

        def process(b, buf, ebuf):
            base = b * RB
            carry = tuple(
                val_v[pl.ds(j * L, L)] for j in range(NV)
            ) + tuple(
                idx_v[pl.ds(j * L, L)] for j in range(NV)
            )

            def grp_body(g, cr, base=base, buf=buf, ebuf=ebuf):
                vals = list(cr[:NV])
                idxs = list(cr[NV:])
                r0 = g * U
                evs = [ebuf[r0 + u, :] for u in range(U)]
                ivs = [jnp.full((L,), base + u, jnp.int32) + r0
                       for u in range(U)]
                for j in range(NV):
                    vcur = [buf[r0 + u, pl.ds(j * L, L)] * evs[u]
                            for u in range(U)]
                    icur = ivs
                    while len(vcur) > 1:
                        nv2, ni2 = [], []
                        for a in range(0, len(vcur), 2):
                            m = vcur[a + 1] > vcur[a]
                            nv2.append(jnp.where(m, vcur[a + 1], vcur[a]))
                            ni2.append(jnp.where(m, icur[a + 1], icur[a]))
                        vcur, icur = nv2, ni2
                    m = vcur[0] > vals[j]
                    vals[j] = jnp.where(m, vcur[0], vals[j])
                    idxs[j] = jnp.where(m, icur[0], idxs[j])
                return tuple(vals) + tuple(idxs)

            carry = lax.fori_loop(0, RB // U, grp_body, carry)
            for j in range(NV):
                val_v[pl.ds(j * L, L)] = carry[j]
                idx_v[pl.ds(j * L, L)] = carry[NV + j]

        def outer(g, acc):
            for k in range(2):
                b = 2 * g + k
                wait(b, k)
                process(b, bufs[k], ebufs[k])

                @pl.when(b + 2 < NBLK)
                def _(b=b, k=k):
                    start(b + 2, k)
            return acc

        lax.fori_loop(0, NBLK // 2, outer, 0)
        if NBLK % 2:
            wait(NBLK - 1, 0)
            process(NBLK - 1, bufs[0], ebufs[0])

        pltpu.sync_copy(val_v, out_emb.at[0, pl.ds(rel0, W)])
        pltpu.sync_copy(idx_v, out_idx.at[pl.ds(rel0, W)])

    return sc_kernel


def _tc_body(e_ref, r_ref, val_ref, idx_ref):
    i = pl.program_id(0)
    prod = r_ref[...] * e_ref[...]
    bmax = jnp.max(prod, axis=0, keepdims=True)
    rows = lax.broadcasted_iota(jnp.int32, (BR, C_TC), 0) + i * BR
    cand = jnp.where(prod == bmax, rows, BIG)
    barg = jnp.min(cand, axis=0, keepdims=True)

    @pl.when(i == 0)
    def _():
        val_ref[...] = jnp.zeros_like(val_ref)
        idx_ref[...] = jnp.zeros_like(idx_ref)

    m = bmax > val_ref[...]
    idx_ref[...] = jnp.where(m, barg, idx_ref[...])
    val_ref[...] = jnp.where(m, bmax, val_ref[...])


_tc_kernel = pl.pallas_call(
    _tc_body,
    grid=(TC_NBLK,),
    in_specs=[
        pl.BlockSpec((BR, 1), lambda i: (i, 0)),
        pl.BlockSpec((BR, C_TC), lambda i: (i, 0)),
    ],
    out_specs=[
        pl.BlockSpec((1, C_TC), lambda i: (0, 0)),
        pl.BlockSpec((1, C_TC), lambda i: (0, 0)),
    ],
    out_shape=[
        jax.ShapeDtypeStruct((1, C_TC), jnp.float32),
        jax.ShapeDtypeStruct((1, C_TC), jnp.int32),
    ],
)

_sc_kernel = _make_sc_kernel()


@jax.jit
def kernel(embedding, r_embedding):
    e_col = embedding.reshape(N, 1)
    e_exp = jnp.broadcast_to(e_col, (N, L))
    sc_val, sc_idx = _sc_kernel(e_exp, r_embedding)
    tc_val, tc_idx = _tc_kernel(e_col, r_embedding)
    new_embedding = jnp.concatenate([tc_val, sc_val], axis=1)
    r_argmax = jnp.concatenate([tc_idx.reshape(C_TC), sc_idx])
    return new_embedding, r_argmax


# speedup vs baseline: 1.0324x; 1.0215x over previous
import functools

import jax
import jax.numpy as jnp
from jax import lax
from jax.experimental import pallas as pl
from jax.experimental.pallas import tpu as pltpu
from jax.experimental.pallas import tpu_sc as plsc

N = 10000
C_TC = 9472
C_SC = N - C_TC

L = 16
NG = 8
NWIN = 4
GR = N // NG
W = 144
NV = W // L
WSTEP = 128
RB = 125
NBLK = GR // RB
U = 5

BR = 400
TC_NBLK = N // BR
BIG = 2 ** 30


def _make_sc_kernel():
    mesh = plsc.VectorSubcoreMesh(core_axis_name="c", subcore_axis_name="s")

    @functools.partial(
        pl.kernel,
        out_type=(
            jax.ShapeDtypeStruct((NG, C_SC), jnp.float32),
            jax.ShapeDtypeStruct((NG, C_SC), jnp.int32),
        ),
        mesh=mesh,
        compiler_params=pltpu.CompilerParams(use_tc_tiling_on_sc=False,
                                             needs_layout_passes=False),
        scratch_types=[
            pltpu.VMEM((RB, W), jnp.float32),
            pltpu.VMEM((RB, W), jnp.float32),
            pltpu.VMEM((RB, L), jnp.float32),
            pltpu.VMEM((RB, L), jnp.float32),
            pltpu.VMEM((W,), jnp.float32),
            pltpu.VMEM((W,), jnp.int32),
            pltpu.SemaphoreType.DMA,
            pltpu.SemaphoreType.DMA,
            pltpu.SemaphoreType.DMA,
            pltpu.SemaphoreType.DMA,
        ],
    )
    def sc_kernel(e_hbm, r_hbm, out_val, out_idx,
                  buf0, buf1, ebuf0, ebuf1, val_v, idx_v,
                  sem0, sem1, esem0, esem1):
        cid = lax.axis_index("c")
        sid = lax.axis_index("s")
        wkr = sid * 2 + cid
        gi = wkr // NWIN
        wi = wkr % NWIN
        r_base = gi * GR
        rel0 = pl.multiple_of(wi * WSTEP, 8)
        c0 = pl.multiple_of(C_TC + wi * WSTEP, 8)

        for j in range(NV):
            val_v[pl.ds(j * L, L)] = jnp.zeros((L,), jnp.float32)
            idx_v[pl.ds(j * L, L)] = jnp.zeros((L,), jnp.int32)

        bufs = (buf0, buf1)
        ebufs = (ebuf0, ebuf1)
        sems = (sem0, sem1)
        esems = (esem0, esem1)

        def start(b, k):
            pltpu.async_copy(
                r_hbm.at[pl.ds(r_base + b * RB, RB), pl.ds(c0, W)],
                bufs[k], sems[k])
            pltpu.async_copy(
                e_hbm.at[pl.ds(r_base + b * RB, RB), :], ebufs[k], esems[k])

        def wait(b, k):
            pltpu.make_async_copy(
                r_hbm.at[pl.ds(r_base + b * RB, RB), pl.ds(c0, W)],
                bufs[k], sems[k]).wait()
            pltpu.make_async_copy(
                e_hbm.at[pl.ds(r_base + b * RB, RB), :], ebufs[k],
                esems[k]).wait()

        start(0, 0)
        start(1, 1)

        def process(b, buf, ebuf):
            base = r_base + b * RB
            carry = tuple(
                val_v[pl.ds(j * L, L)] for j in range(NV)
            ) + tuple(
                idx_v[pl.ds(j * L, L)] for j in range(NV)
            )

            def grp_body(g, cr, base=base, buf=buf, ebuf=ebuf):
                vals = list(cr[:NV])
                idxs = list(cr[NV:])
                r0 = g * U
                evs = [ebuf[r0 + u, :] for u in range(U)]
                ivs = [jnp.full((L,), base + u, jnp.int32) + r0
                       for u in range(U)]
                for j in range(NV):
                    vcur = [buf[r0 + u, pl.ds(j * L, L)] * evs[u]
                            for u in range(U)]
                    icur = ivs
                    while len(vcur) > 1:
                        nv2, ni2 = [], []
                        for a in range(0, len(vcur) - 1, 2):
                            m = vcur[a + 1] > vcur[a]
                            nv2.append(jnp.where(m, vcur[a + 1], vcur[a]))
                            ni2.append(jnp.where(m, icur[a + 1], icur[a]))
                        if len(vcur) % 2:
                            nv2.append(vcur[-1])
                            ni2.append(icur[-1])
                        vcur, icur = nv2, ni2
                    m = vcur[0] > vals[j]
                    vals[j] = jnp.where(m, vcur[0], vals[j])
                    idxs[j] = jnp.where(m, icur[0], idxs[j])
                return tuple(vals) + tuple(idxs)

            carry = lax.fori_loop(0, RB // U, grp_body, carry)
            for j in range(NV):
                val_v[pl.ds(j * L, L)] = carry[j]
                idx_v[pl.ds(j * L, L)] = carry[NV + j]

        def outer(g, acc):
            for k in range(2):
                b = 2 * g + k
                wait(b, k)
                process(b, bufs[k], ebufs[k])

                @pl.when(b + 2 < NBLK)
                def _(b=b, k=k):
                    start(b + 2, k)
            return acc

        lax.fori_loop(0, NBLK // 2, outer, 0)
        if NBLK % 2:
            wait(NBLK - 1, 0)
            process(NBLK - 1, bufs[0], ebufs[0])

        pltpu.sync_copy(val_v, out_val.at[gi, pl.ds(rel0, W)])
        pltpu.sync_copy(idx_v, out_idx.at[gi, pl.ds(rel0, W)])

    return sc_kernel


def _tc_body(e_ref, r_ref, val_ref, idx_ref):
    i = pl.program_id(0)
    prod = r_ref[...] * e_ref[...]
    bmax = jnp.max(prod, axis=0, keepdims=True)
    rows = lax.broadcasted_iota(jnp.int32, (BR, C_TC), 0) + i * BR
    cand = jnp.where(prod == bmax, rows, BIG)
    barg = jnp.min(cand, axis=0, keepdims=True)

    @pl.when(i == 0)
    def _():
        val_ref[...] = jnp.zeros_like(val_ref)
        idx_ref[...] = jnp.zeros_like(idx_ref)

    m = bmax > val_ref[...]
    idx_ref[...] = jnp.where(m, barg, idx_ref[...])
    val_ref[...] = jnp.where(m, bmax, val_ref[...])


_tc_kernel = pl.pallas_call(
    _tc_body,
    grid=(TC_NBLK,),
    in_specs=[
        pl.BlockSpec((BR, 1), lambda i: (i, 0)),
        pl.BlockSpec((BR, C_TC), lambda i: (i, 0)),
    ],
    out_specs=[
        pl.BlockSpec((1, C_TC), lambda i: (0, 0)),
        pl.BlockSpec((1, C_TC), lambda i: (0, 0)),
    ],
    out_shape=[
        jax.ShapeDtypeStruct((1, C_TC), jnp.float32),
        jax.ShapeDtypeStruct((1, C_TC), jnp.int32),
    ],
)


def _merge_body(pv_ref, pi_ref, val_ref, idx_ref):
    v = pv_ref[0:1, :]
    x = pi_ref[0:1, :]
    for g in range(1, NG):
        gv = pv_ref[g:g + 1, :]
        gx = pi_ref[g:g + 1, :]
        m = gv > v
        v = jnp.where(m, gv, v)
        x = jnp.where(m, gx, x)
    val_ref[...] = v
    idx_ref[...] = x


_merge_kernel = pl.pallas_call(
    _merge_body,
    out_shape=[
        jax.ShapeDtypeStruct((1, C_SC), jnp.float32),
        jax.ShapeDtypeStruct((1, C_SC), jnp.int32),
    ],
)

_sc_kernel = _make_sc_kernel()


@jax.jit
def kernel(embedding, r_embedding):
    e_col = embedding.reshape(N, 1)
    e_exp = jnp.broadcast_to(e_col, (N, L))
    sc_pv, sc_pi = _sc_kernel(e_exp, r_embedding)
    tc_val, tc_idx = _tc_kernel(e_col, r_embedding)
    sc_val, sc_idx = _merge_kernel(sc_pv, sc_pi)
    new_embedding = jnp.concatenate([tc_val, sc_val], axis=1)
    r_argmax = jnp.concatenate([tc_idx.reshape(C_TC), sc_idx.reshape(C_SC)])
    return new_embedding, r_argmax
